# baseline (speedup 1.0000x reference)
import functools

import jax
import jax.numpy as jnp
from jax import lax
from jax.experimental import pallas as pl
from jax.experimental.pallas import tpu as pltpu
from jax.experimental.pallas import tpu_sc as plsc

_B, _N, _K = 8, 4096, 32
_NC, _NS, _L = 2, 16, 16
_NW = _NC * _NS
_NPW = _B * _N // _NW
_CH = 128
_NCHUNK = _NPW // _CH
_G = _CH // _L
_PARTS = _NW // _B


def _bf16r(x):
    i = lax.bitcast_convert_type(x, jnp.int32)
    i = jnp.bitwise_and(i + jnp.int32(0x8000), jnp.int32(-65536))
    return lax.bitcast_convert_type(i, jnp.float32)


def _rsqrt(x):
    i = lax.bitcast_convert_type(x, jnp.int32)
    i = jnp.int32(0x5F3759DF) - jnp.right_shift(i, jnp.ones_like(i))
    y = lax.bitcast_convert_type(i, jnp.float32)
    for _ in range(3):
        y = y * (1.5 - 0.5 * x * y * y)
    return y


def _proj_so3(m):
    def dot3(a, b):
        return a[0] * b[0] + a[1] * b[1] + a[2] * b[2]

    col = lambda j: [m[j], m[3 + j], m[6 + j]]
    c0, c1, c2 = col(0), col(1), col(2)
    S = {
        (0, 0): dot3(c0, c0), (1, 1): dot3(c1, c1), (2, 2): dot3(c2, c2),
        (0, 1): dot3(c0, c1), (0, 2): dot3(c0, c2), (1, 2): dot3(c1, c2),
    }
    one = jnp.ones_like(S[(0, 0)])
    zero = jnp.zeros_like(S[(0, 0)])
    V = [[one, zero, zero], [zero, one, zero], [zero, zero, one]]

    def getS(i, j):
        return S[(i, j)] if i <= j else S[(j, i)]

    for _ in range(3):
        for (p, q) in ((0, 1), (0, 2), (1, 2)):
            app, aqq, apq = getS(p, p), getS(q, q), getS(p, q)
            small = jnp.abs(apq) < 1e-30
            denom = jnp.where(small, one, 2.0 * apq)
            theta = (aqq - app) / denom
            t2 = theta * theta + 1.0
            rt = t2 * _rsqrt(t2)
            sgn = jnp.sign(theta)
            t = jnp.where(sgn == 0.0, one, sgn) / (jnp.abs(theta) + rt)
            c = _rsqrt(1.0 + t * t)
            s = t * c
            c = jnp.where(small, one, c)
            s = jnp.where(small, zero, s)
            r = 3 - p - q
            arp, arq = getS(r, p), getS(r, q)
            S[(p, p)] = c * c * app - 2.0 * s * c * apq + s * s * aqq
            S[(q, q)] = s * s * app + 2.0 * s * c * apq + c * c * aqq
            S[(p, q)] = zero
            S[(r, p) if r <= p else (p, r)] = c * arp - s * arq
            S[(r, q) if r <= q else (q, r)] = s * arp + c * arq
            for row in range(3):
                vp, vq = V[row][p], V[row][q]
                V[row][p] = c * vp - s * vq
                V[row][q] = s * vp + c * vq

    l0, l1, l2 = getS(0, 0), getS(1, 1), getS(2, 2)
    colV = lambda j: [V[0][j], V[1][j], V[2][j]]
    v0, v1, v2 = colV(0), colV(1), colV(2)
    is_min0 = (l0 <= l1) & (l0 <= l2)
    is_min2 = (~is_min0) & ((l2 <= l0) & (l2 <= l1))
    sel = lambda mask, a, b: [jnp.where(mask, x, y) for x, y in zip(a, b)]
    va = sel(is_min0, v1, v0)
    vb = sel(is_min2, v1, v2)

    def matvec(v):
        return [
            m[0] * v[0] + m[1] * v[1] + m[2] * v[2],
            m[3] * v[0] + m[4] * v[1] + m[5] * v[2],
            m[6] * v[0] + m[7] * v[1] + m[8] * v[2],
        ]

    ba, bb = matvec(va), matvec(vb)
    ua = [x * _rsqrt(dot3(ba, ba)) for x in ba]
    proj = dot3(ua, bb)
    ub = [x - proj * u for x, u in zip(bb, ua)]
    ub = [x * _rsqrt(dot3(ub, ub)) for x in ub]

    def cross(a, b):
        return [
            a[1] * b[2] - a[2] * b[1],
            a[2] * b[0] - a[0] * b[2],
            a[0] * b[1] - a[1] * b[0],
        ]

    uc = cross(ua, ub)
    wc = cross(va, vb)
    return [
        ua[r] * va[c] + ub[r] * vb[c] + uc[r] * wc[c]
        for r in range(3)
        for c in range(3)
    ]


@functools.partial(
    pl.kernel,
    mesh=plsc.VectorSubcoreMesh(core_axis_name="c", subcore_axis_name="s"),
    compiler_params=pltpu.CompilerParams(needs_layout_passes=False),
    out_type=(
        jax.ShapeDtypeStruct((_B * _N * 9,), jnp.float32),
        jax.ShapeDtypeStruct((_B * _N * 3,), jnp.float32),
    ),
    scratch_types=[
        pltpu.VMEM((_N * 9,), jnp.float32),
        pltpu.VMEM((_N * 3,), jnp.float32),
        pltpu.VMEM((_CH * _K * 9,), jnp.float32),
        pltpu.VMEM((_CH * _K * 3,), jnp.float32),
        pltpu.VMEM((_CH * _K,), jnp.float32),
        pltpu.VMEM((_CH * _K,), jnp.int32),
        pltpu.VMEM((_CH * 9,), jnp.float32),
        pltpu.VMEM((_CH * 3,), jnp.float32),
    ],
)
def _sc_solver(rot_hbm, trans_hbm, prot_hbm, ptrans_hbm, conf_hbm, topo_hbm,
               orot_hbm, otrans_hbm,
               rot_v, trans_v, prot_v, ptrans_v, conf_v, topo_v,
               orot_v, otrans_v):
    wid = lax.axis_index("s") * _NC + lax.axis_index("c")
    b = wid // _PARTS
    part = wid % _PARTS
    node0 = b * _N + part * _NPW

    pltpu.sync_copy(rot_hbm.at[pl.ds(b * _N * 9, _N * 9)], rot_v)
    pltpu.sync_copy(trans_hbm.at[pl.ds(b * _N * 3, _N * 3)], trans_v)

    def round_body(i, carry):
        s = pl.ds(i * _L, _L)
        rot_v[s] = _bf16r(rot_v[s])
        return carry

    lax.fori_loop(0, _N * 9 // _L, round_body, 0)

    def chunk_body(ci, carry):
        gbase = node0 + ci * _CH
        pltpu.sync_copy(prot_hbm.at[pl.ds(gbase * _K * 9, _CH * _K * 9)], prot_v)
        pltpu.sync_copy(ptrans_hbm.at[pl.ds(gbase * _K * 3, _CH * _K * 3)], ptrans_v)
        pltpu.sync_copy(conf_hbm.at[pl.ds(gbase * _K, _CH * _K)], conf_v)
        pltpu.sync_copy(topo_hbm.at[pl.ds(gbase * _K, _CH * _K)], topo_v)

        def group_body(g, carry2):
            lane = lax.iota(jnp.int32, _L) + g * _L

            def k_body(k, acc):
                a9, a3, wsum = acc[:9], acc[9:12], acc[12]
                e = lane * _K + k
                t = plsc.load_gather(topo_v, [e])
                w = plsc.load_gather(conf_v, [e])
                t9 = t * 9
                t3 = t * 3
                e9 = e * 9
                e3 = e * 3
                Rj = [plsc.load_gather(rot_v, [t9 + j]) for j in range(9)]
                tj = [plsc.load_gather(trans_v, [t3 + j]) for j in range(3)]
                pr = [_bf16r(plsc.load_gather(prot_v, [e9 + j])) for j in range(9)]
                pt = [_bf16r(plsc.load_gather(ptrans_v, [e3 + j])) for j in range(3)]
                na9 = []
                for r in range(3):
                    for c in range(3):
                        comp = (Rj[r * 3] * pr[c] + Rj[r * 3 + 1] * pr[3 + c]
                                + Rj[r * 3 + 2] * pr[6 + c])
                        na9.append(a9[r * 3 + c] + w * comp)
                na3 = []
                for r in range(3):
                    ct = (Rj[r * 3] * pt[0] + Rj[r * 3 + 1] * pt[1]
                          + Rj[r * 3 + 2] * pt[2] + tj[r])
                    na3.append(a3[r] + w * ct)
                return tuple(na9) + tuple(na3) + (wsum + w,)

            zero = jnp.zeros((_L,), jnp.float32)
            init = tuple(zero for _ in range(12)) + (zero,)
            acc = lax.fori_loop(0, _K, k_body, init, unroll=8)
            inv = 1.0 / acc[12]
            avg = [a * inv for a in acc[:9]]
            avgt = [a * inv for a in acc[9:12]]
            R = _proj_so3(avg)
            l9 = lane * 9
            l3 = lane * 3
            for j in range(9):
                plsc.store_scatter(orot_v, [l9 + j], R[j])
            for j in range(3):
                plsc.store_scatter(otrans_v, [l3 + j], avgt[j])
            return carry2

        lax.fori_loop(0, _G, group_body, 0)
        pltpu.sync_copy(orot_v, orot_hbm.at[pl.ds(gbase * 9, _CH * 9)])
        pltpu.sync_copy(otrans_v, otrans_hbm.at[pl.ds(gbase * 3, _CH * 3)])
        return carry

    lax.fori_loop(0, _NCHUNK, chunk_body, 0)


def kernel(rot, trans, pair_rot, pair_trans, confidences, topology):
    rot_f = rot.reshape(-1)
    trans_f = trans.reshape(-1)
    prot_f = pair_rot.reshape(-1)
    ptrans_f = pair_trans.reshape(-1)
    conf_f = confidences.reshape(-1)
    topo_f = topology.astype(jnp.int32).reshape(-1)
    orot, otrans = _sc_solver(rot_f, trans_f, prot_f, ptrans_f, conf_f, topo_f)
    return orot.reshape(_B, _N, 3, 3), otrans.reshape(_B, _N, 3)


# speedup vs baseline: 14.5566x; 14.5566x over previous
import functools

import jax
import jax.numpy as jnp
from jax import lax
from jax.experimental import pallas as pl
from jax.experimental.pallas import tpu as pltpu
from jax.experimental.pallas import tpu_sc as plsc

_B, _N, _K = 8, 4096, 32
_NC, _NS, _L = 2, 16, 16
_NW = _NC * _NS
_NPW = _B * _N // _NW
_CH = 128
_NCHUNK = _NPW // _CH
_G = _CH // _L
_PARTS = _NW // _B


def _bf16r(x):
    i = lax.bitcast_convert_type(x, jnp.int32)
    i = jnp.bitwise_and(i + jnp.int32(0x8000), jnp.int32(-65536))
    return lax.bitcast_convert_type(i, jnp.float32)


def _rsqrt(x):
    i = lax.bitcast_convert_type(x, jnp.int32)
    i = jnp.int32(0x5F3759DF) - jnp.right_shift(i, jnp.ones_like(i))
    y = lax.bitcast_convert_type(i, jnp.float32)
    for _ in range(3):
        y = y * (1.5 - 0.5 * x * y * y)
    return y


def _proj_so3(m):
    def dot3(a, b):
        return a[0] * b[0] + a[1] * b[1] + a[2] * b[2]

    col = lambda j: [m[j], m[3 + j], m[6 + j]]
    c0, c1, c2 = col(0), col(1), col(2)
    S = {
        (0, 0): dot3(c0, c0), (1, 1): dot3(c1, c1), (2, 2): dot3(c2, c2),
        (0, 1): dot3(c0, c1), (0, 2): dot3(c0, c2), (1, 2): dot3(c1, c2),
    }
    one = jnp.ones_like(S[(0, 0)])
    zero = jnp.zeros_like(S[(0, 0)])
    V = [[one, zero, zero], [zero, one, zero], [zero, zero, one]]

    def getS(i, j):
        return S[(i, j)] if i <= j else S[(j, i)]

    for _ in range(3):
        for (p, q) in ((0, 1), (0, 2), (1, 2)):
            app, aqq, apq = getS(p, p), getS(q, q), getS(p, q)
            small = jnp.abs(apq) < 1e-30
            denom = jnp.where(small, one, 2.0 * apq)
            theta = (aqq - app) / denom
            t2 = theta * theta + 1.0
            rt = t2 * _rsqrt(t2)
            sgn = jnp.sign(theta)
            t = jnp.where(sgn == 0.0, one, sgn) / (jnp.abs(theta) + rt)
            c = _rsqrt(1.0 + t * t)
            s = t * c
            c = jnp.where(small, one, c)
            s = jnp.where(small, zero, s)
            r = 3 - p - q
            arp, arq = getS(r, p), getS(r, q)
            S[(p, p)] = c * c * app - 2.0 * s * c * apq + s * s * aqq
            S[(q, q)] = s * s * app + 2.0 * s * c * apq + c * c * aqq
            S[(p, q)] = zero
            S[(r, p) if r <= p else (p, r)] = c * arp - s * arq
            S[(r, q) if r <= q else (q, r)] = s * arp + c * arq
            for row in range(3):
                vp, vq = V[row][p], V[row][q]
                V[row][p] = c * vp - s * vq
                V[row][q] = s * vp + c * vq

    l0, l1, l2 = getS(0, 0), getS(1, 1), getS(2, 2)
    colV = lambda j: [V[0][j], V[1][j], V[2][j]]
    v0, v1, v2 = colV(0), colV(1), colV(2)
    is_min0 = (l0 <= l1) & (l0 <= l2)
    is_min2 = (~is_min0) & ((l2 <= l0) & (l2 <= l1))
    sel = lambda mask, a, b: [jnp.where(mask, x, y) for x, y in zip(a, b)]
    va = sel(is_min0, v1, v0)
    vb = sel(is_min2, v1, v2)

    def matvec(v):
        return [
            m[0] * v[0] + m[1] * v[1] + m[2] * v[2],
            m[3] * v[0] + m[4] * v[1] + m[5] * v[2],
            m[6] * v[0] + m[7] * v[1] + m[8] * v[2],
        ]

    ba, bb = matvec(va), matvec(vb)
    ua = [x * _rsqrt(dot3(ba, ba)) for x in ba]
    proj = dot3(ua, bb)
    ub = [x - proj * u for x, u in zip(bb, ua)]
    ub = [x * _rsqrt(dot3(ub, ub)) for x in ub]

    def cross(a, b):
        return [
            a[1] * b[2] - a[2] * b[1],
            a[2] * b[0] - a[0] * b[2],
            a[0] * b[1] - a[1] * b[0],
        ]

    uc = cross(ua, ub)
    wc = cross(va, vb)
    return [
        ua[r] * va[c] + ub[r] * vb[c] + uc[r] * wc[c]
        for r in range(3)
        for c in range(3)
    ]


@functools.partial(
    pl.kernel,
    mesh=plsc.VectorSubcoreMesh(core_axis_name="c", subcore_axis_name="s"),
    compiler_params=pltpu.CompilerParams(
        needs_layout_passes=False, use_tc_tiling_on_sc=True),
    out_type=(
        jax.ShapeDtypeStruct((_B * _N * 9,), jnp.float32),
        jax.ShapeDtypeStruct((_B * _N * 3,), jnp.float32),
    ),
    scratch_types=(
        [pltpu.VMEM((_N,), jnp.float32) for _ in range(9)]
        + [pltpu.VMEM((_N,), jnp.float32) for _ in range(3)]
        + [pltpu.VMEM((_K, _CH), jnp.float32) for _ in range(9)]
        + [pltpu.VMEM((_K, _CH), jnp.float32) for _ in range(3)]
        + [
            pltpu.VMEM((_K, _CH), jnp.float32),
            pltpu.VMEM((_K, _CH), jnp.int32),
            pltpu.VMEM((_CH * 9,), jnp.float32),
            pltpu.VMEM((_CH * 3,), jnp.float32),
        ]
    ),
)
def _sc_solver(rot_hbm, trans_hbm, prot_hbm, ptrans_hbm, conf_hbm, topo_hbm,
               orot_hbm, otrans_hbm, *scratch):
    rot_v = scratch[0:9]
    trans_v = scratch[9:12]
    prot_v = scratch[12:21]
    ptrans_v = scratch[21:24]
    conf_v, topo_v, orot_v, otrans_v = scratch[24:28]

    wid = lax.axis_index("s") * _NC + lax.axis_index("c")
    b = wid // _PARTS
    part = wid % _PARTS
    nbase = part * _NPW
    node0 = b * _N + nbase

    for r in range(3):
        for c in range(3):
            pltpu.sync_copy(rot_hbm.at[r, c, b], rot_v[r * 3 + c])
    for c in range(3):
        pltpu.sync_copy(trans_hbm.at[c, b], trans_v[c])

    for j in range(9):
        def round_body(i, carry, _ref=rot_v[j]):
            s = pl.ds(i * _L, _L)
            _ref[s] = _bf16r(_ref[s])
            return carry

        lax.fori_loop(0, _N // _L, round_body, 0)

    def chunk_body(ci, carry):
        nc = nbase + ci * _CH
        for r in range(3):
            for c in range(3):
                pltpu.sync_copy(prot_hbm.at[b, r, c, :, pl.ds(nc, _CH)],
                                prot_v[r * 3 + c])
        for c in range(3):
            pltpu.sync_copy(ptrans_hbm.at[b, c, :, pl.ds(nc, _CH)],
                            ptrans_v[c])
        pltpu.sync_copy(conf_hbm.at[pl.ds(b * _K, _K), ci + part * _NCHUNK],
                        conf_v)
        pltpu.sync_copy(topo_hbm.at[b, :, pl.ds(nc, _CH)], topo_v)

        def group_body(g, carry2):
            nn = g * _L

            def k_body(k, acc):
                a9, a3, wsum = acc[:9], acc[9:12], acc[12]
                sl = pl.ds(nn, _L)
                t = topo_v[k, sl]
                w = conf_v[k, sl]
                Rj = [plsc.load_gather(rot_v[j], [t]) for j in range(9)]
                tj = [plsc.load_gather(trans_v[j], [t]) for j in range(3)]
                pr = [_bf16r(prot_v[j][k, sl]) for j in range(9)]
                pt = [_bf16r(ptrans_v[j][k, sl]) for j in range(3)]
                na9 = []
                for r in range(3):
                    for c in range(3):
                        comp = (Rj[r * 3] * pr[c] + Rj[r * 3 + 1] * pr[3 + c]
                                + Rj[r * 3 + 2] * pr[6 + c])
                        na9.append(a9[r * 3 + c] + w * comp)
                na3 = []
                for r in range(3):
                    ct = (Rj[r * 3] * pt[0] + Rj[r * 3 + 1] * pt[1]
                          + Rj[r * 3 + 2] * pt[2] + tj[r])
                    na3.append(a3[r] + w * ct)
                return tuple(na9) + tuple(na3) + (wsum + w,)

            zero = jnp.zeros((_L,), jnp.float32)
            init = tuple(zero for _ in range(12)) + (zero,)
            acc = lax.fori_loop(0, _K, k_body, init, unroll=4)
            inv = 1.0 / acc[12]
            avg = [a * inv for a in acc[:9]]
            avgt = [a * inv for a in acc[9:12]]
            R = _proj_so3(avg)
            lane = lax.iota(jnp.int32, _L) + nn
            l9 = lane * 9
            l3 = lane * 3
            for j in range(9):
                plsc.store_scatter(orot_v, [l9 + j], R[j])
            for j in range(3):
                plsc.store_scatter(otrans_v, [l3 + j], avgt[j])
            return carry2

        lax.fori_loop(0, _G, group_body, 0)
        gbase = node0 + ci * _CH
        pltpu.sync_copy(orot_v, orot_hbm.at[pl.ds(gbase * 9, _CH * 9)])
        pltpu.sync_copy(otrans_v, otrans_hbm.at[pl.ds(gbase * 3, _CH * 3)])
        return carry

    lax.fori_loop(0, _NCHUNK, chunk_body, 0)


def kernel(rot, trans, pair_rot, pair_trans, confidences, topology):
    rot_t = jnp.transpose(rot, (2, 3, 0, 1))
    trans_t = jnp.transpose(trans, (2, 0, 1))
    prot_t = jnp.transpose(pair_rot, (0, 3, 4, 2, 1))
    ptrans_t = jnp.transpose(pair_trans, (0, 3, 2, 1))
    conf_t = jnp.transpose(confidences, (0, 2, 3, 1)).reshape(
        _B * _K, _N // _CH, _CH)
    topo_t = jnp.transpose(topology.astype(jnp.int32), (0, 2, 1))
    orot, otrans = _sc_solver(rot_t, trans_t, prot_t, ptrans_t, conf_t, topo_t)
    return orot.reshape(_B, _N, 3, 3), otrans.reshape(_B, _N, 3)


# speedup vs baseline: 19.4925x; 1.3391x over previous
import functools

import jax
import jax.numpy as jnp
from jax import lax
from jax.experimental import pallas as pl
from jax.experimental.pallas import tpu as pltpu
from jax.experimental.pallas import tpu_sc as plsc

_B, _N, _K = 8, 4096, 32
_NC, _NS, _L = 2, 16, 16
_NW = _NC * _NS
_NPW = _B * _N // _NW
_CH = 128
_NCHUNK = _NPW // _CH
_G = _CH // _L
_PARTS = _NW // _B


def _bf16r(x):
    i = lax.bitcast_convert_type(x, jnp.int32)
    i = jnp.bitwise_and(i + jnp.int32(0x8000), jnp.int32(-65536))
    return lax.bitcast_convert_type(i, jnp.float32)


def _rsqrt(x):
    i = lax.bitcast_convert_type(x, jnp.int32)
    i = jnp.int32(0x5F3759DF) - jnp.right_shift(i, jnp.ones_like(i))
    y = lax.bitcast_convert_type(i, jnp.float32)
    for _ in range(3):
        y = y * (1.5 - 0.5 * x * y * y)
    return y


def _proj_so3(m):
    def dot3(a, b):
        return a[0] * b[0] + a[1] * b[1] + a[2] * b[2]

    col = lambda j: [m[j], m[3 + j], m[6 + j]]
    c0, c1, c2 = col(0), col(1), col(2)
    S = {
        (0, 0): dot3(c0, c0), (1, 1): dot3(c1, c1), (2, 2): dot3(c2, c2),
        (0, 1): dot3(c0, c1), (0, 2): dot3(c0, c2), (1, 2): dot3(c1, c2),
    }
    one = jnp.ones_like(S[(0, 0)])
    zero = jnp.zeros_like(S[(0, 0)])
    V = [[one, zero, zero], [zero, one, zero], [zero, zero, one]]

    def getS(i, j):
        return S[(i, j)] if i <= j else S[(j, i)]

    for _ in range(3):
        for (p, q) in ((0, 1), (0, 2), (1, 2)):
            app, aqq, apq = getS(p, p), getS(q, q), getS(p, q)
            small = jnp.abs(apq) < 1e-30
            denom = jnp.where(small, one, 2.0 * apq)
            theta = (aqq - app) / denom
            t2 = theta * theta + 1.0
            rt = t2 * _rsqrt(t2)
            sgn = jnp.sign(theta)
            t = jnp.where(sgn == 0.0, one, sgn) / (jnp.abs(theta) + rt)
            c = _rsqrt(1.0 + t * t)
            s = t * c
            c = jnp.where(small, one, c)
            s = jnp.where(small, zero, s)
            r = 3 - p - q
            arp, arq = getS(r, p), getS(r, q)
            S[(p, p)] = c * c * app - 2.0 * s * c * apq + s * s * aqq
            S[(q, q)] = s * s * app + 2.0 * s * c * apq + c * c * aqq
            S[(p, q)] = zero
            S[(r, p) if r <= p else (p, r)] = c * arp - s * arq
            S[(r, q) if r <= q else (q, r)] = s * arp + c * arq
            for row in range(3):
                vp, vq = V[row][p], V[row][q]
                V[row][p] = c * vp - s * vq
                V[row][q] = s * vp + c * vq

    l0, l1, l2 = getS(0, 0), getS(1, 1), getS(2, 2)
    colV = lambda j: [V[0][j], V[1][j], V[2][j]]
    v0, v1, v2 = colV(0), colV(1), colV(2)
    is_min0 = (l0 <= l1) & (l0 <= l2)
    is_min2 = (~is_min0) & ((l2 <= l0) & (l2 <= l1))
    sel = lambda mask, a, b: [jnp.where(mask, x, y) for x, y in zip(a, b)]
    va = sel(is_min0, v1, v0)
    vb = sel(is_min2, v1, v2)

    def matvec(v):
        return [
            m[0] * v[0] + m[1] * v[1] + m[2] * v[2],
            m[3] * v[0] + m[4] * v[1] + m[5] * v[2],
            m[6] * v[0] + m[7] * v[1] + m[8] * v[2],
        ]

    ba, bb = matvec(va), matvec(vb)
    ua = [x * _rsqrt(dot3(ba, ba)) for x in ba]
    proj = dot3(ua, bb)
    ub = [x - proj * u for x, u in zip(bb, ua)]
    ub = [x * _rsqrt(dot3(ub, ub)) for x in ub]

    def cross(a, b):
        return [
            a[1] * b[2] - a[2] * b[1],
            a[2] * b[0] - a[0] * b[2],
            a[0] * b[1] - a[1] * b[0],
        ]

    uc = cross(ua, ub)
    wc = cross(va, vb)
    return [
        ua[r] * va[c] + ub[r] * vb[c] + uc[r] * wc[c]
        for r in range(3)
        for c in range(3)
    ]


@functools.partial(
    pl.kernel,
    mesh=plsc.VectorSubcoreMesh(core_axis_name="c", subcore_axis_name="s"),
    compiler_params=pltpu.CompilerParams(
        needs_layout_passes=False, use_tc_tiling_on_sc=True),
    out_type=(
        jax.ShapeDtypeStruct((9 * _N // _CH, _B, _CH), jnp.float32),
        jax.ShapeDtypeStruct((3 * _N // _CH, _B, _CH), jnp.float32),
    ),
    scratch_types=(
        [pltpu.VMEM((_N,), jnp.float32) for _ in range(9)]
        + [pltpu.VMEM((_N,), jnp.float32) for _ in range(3)]
        + [pltpu.VMEM((_K, _CH), jnp.float32) for _ in range(9)]
        + [pltpu.VMEM((_K, _CH), jnp.float32) for _ in range(3)]
        + [
            pltpu.VMEM((_K, _CH), jnp.float32),
            pltpu.VMEM((_K, _CH), jnp.int32),
            pltpu.VMEM((9, _NCHUNK, _CH), jnp.float32),
            pltpu.VMEM((3, _NCHUNK, _CH), jnp.float32),
        ]
    ),
)
def _sc_solver(rot_hbm, trans_hbm, prot_hbm, ptrans_hbm, conf_hbm, topo_hbm,
               orot_hbm, otrans_hbm, *scratch):
    rot_v = scratch[0:9]
    trans_v = scratch[9:12]
    prot_v = scratch[12:21]
    ptrans_v = scratch[21:24]
    conf_v, topo_v, orot_v, otrans_v = scratch[24:28]

    wid = lax.axis_index("s") * _NC + lax.axis_index("c")
    b = wid // _PARTS
    part = wid % _PARTS
    nbase = part * _NPW
    node0 = b * _N + nbase

    for r in range(3):
        for c in range(3):
            pltpu.sync_copy(rot_hbm.at[r, c, b], rot_v[r * 3 + c])
    for c in range(3):
        pltpu.sync_copy(trans_hbm.at[c, b], trans_v[c])

    for j in range(9):
        def round_body(i, carry, _ref=rot_v[j]):
            s = pl.ds(i * _L, _L)
            _ref[s] = _bf16r(_ref[s])
            return carry

        lax.fori_loop(0, _N // _L, round_body, 0)

    def chunk_body(ci, carry):
        nc = nbase + ci * _CH
        for r in range(3):
            for c in range(3):
                pltpu.sync_copy(prot_hbm.at[b, r, c, :, pl.ds(nc, _CH)],
                                prot_v[r * 3 + c])
        for c in range(3):
            pltpu.sync_copy(ptrans_hbm.at[b, c, :, pl.ds(nc, _CH)],
                            ptrans_v[c])
        pltpu.sync_copy(conf_hbm.at[pl.ds(b * _K, _K), ci + part * _NCHUNK],
                        conf_v)
        pltpu.sync_copy(topo_hbm.at[b, :, pl.ds(nc, _CH)], topo_v)

        def group_body(g, carry2):
            nn = g * _L

            def k_body(k, acc):
                a9, a3, wsum = acc[:9], acc[9:12], acc[12]
                sl = pl.ds(nn, _L)
                t = topo_v[k, sl]
                w = conf_v[k, sl]
                Rj = [plsc.load_gather(rot_v[j], [t]) for j in range(9)]
                tj = [plsc.load_gather(trans_v[j], [t]) for j in range(3)]
                pr = [_bf16r(prot_v[j][k, sl]) for j in range(9)]
                pt = [_bf16r(ptrans_v[j][k, sl]) for j in range(3)]
                na9 = []
                for r in range(3):
                    for c in range(3):
                        comp = (Rj[r * 3] * pr[c] + Rj[r * 3 + 1] * pr[3 + c]
                                + Rj[r * 3 + 2] * pr[6 + c])
                        na9.append(a9[r * 3 + c] + w * comp)
                na3 = []
                for r in range(3):
                    ct = (Rj[r * 3] * pt[0] + Rj[r * 3 + 1] * pt[1]
                          + Rj[r * 3 + 2] * pt[2] + tj[r])
                    na3.append(a3[r] + w * ct)
                return tuple(na9) + tuple(na3) + (wsum + w,)

            zero = jnp.zeros((_L,), jnp.float32)
            init = tuple(zero for _ in range(12)) + (zero,)
            acc = lax.fori_loop(0, _K, k_body, init, unroll=4)
            inv = 1.0 / acc[12]
            avg = [a * inv for a in acc[:9]]
            avgt = [a * inv for a in acc[9:12]]
            R = _proj_so3(avg)
            sl16 = pl.ds(nn, _L)
            for j in range(9):
                orot_v[j, ci, sl16] = R[j]
            for j in range(3):
                otrans_v[j, ci, sl16] = avgt[j]
            return carry2

        lax.fori_loop(0, _G, group_body, 0)
        return carry

    lax.fori_loop(0, _NCHUNK, chunk_body, 0)

    row0 = part * _NCHUNK
    for j in range(9):
        pltpu.sync_copy(orot_v.at[j],
                        orot_hbm.at[pl.ds(j * (_N // _CH) + row0, _NCHUNK), b])
    for j in range(3):
        pltpu.sync_copy(otrans_v.at[j],
                        otrans_hbm.at[pl.ds(j * (_N // _CH) + row0, _NCHUNK), b])


def kernel(rot, trans, pair_rot, pair_trans, confidences, topology):
    rot_t = jnp.transpose(rot, (2, 3, 0, 1))
    trans_t = jnp.transpose(trans, (2, 0, 1))
    prot_t = jnp.transpose(pair_rot, (0, 3, 4, 2, 1))
    ptrans_t = jnp.transpose(pair_trans, (0, 3, 2, 1))
    conf_t = jnp.transpose(confidences, (0, 2, 3, 1)).reshape(
        _B * _K, _N // _CH, _CH)
    topo_t = jnp.transpose(topology.astype(jnp.int32), (0, 2, 1))
    orot, otrans = _sc_solver(rot_t, trans_t, prot_t, ptrans_t, conf_t, topo_t)
    out_rot = orot.reshape(3, 3, _N // _CH, _B, _CH).transpose(
        (3, 2, 4, 0, 1)).reshape(_B, _N, 3, 3)
    out_trans = otrans.reshape(3, _N // _CH, _B, _CH).transpose(
        (2, 1, 3, 0)).reshape(_B, _N, 3)
    return out_rot, out_trans


# speedup vs baseline: 26.1740x; 1.3428x over previous
import functools

import jax
import jax.numpy as jnp
from jax import lax
from jax.experimental import pallas as pl
from jax.experimental.pallas import tpu as pltpu
from jax.experimental.pallas import tpu_sc as plsc

_B, _N, _K = 8, 4096, 32
_NC, _NS, _L = 2, 16, 16
_NW = _NC * _NS
_NPW = _B * _N // _NW
_CH = 128
_NCHUNK = _NPW // _CH
_G = _CH // _L
_KH = _K // 2
_PARTS = _NW // _B
_NT = _N // 128


def _bf16r(x):
    i = lax.bitcast_convert_type(x, jnp.int32)
    i = jnp.bitwise_and(i + jnp.int32(0x8000), jnp.int32(-65536))
    return lax.bitcast_convert_type(i, jnp.float32)


def _rsqrt(x):
    i = lax.bitcast_convert_type(x, jnp.int32)
    i = jnp.int32(0x5F3759DF) - jnp.right_shift(i, jnp.ones_like(i))
    y = lax.bitcast_convert_type(i, jnp.float32)
    for _ in range(3):
        y = y * (1.5 - 0.5 * x * y * y)
    return y


def _proj_so3(m):
    def dot3(a, b):
        return a[0] * b[0] + a[1] * b[1] + a[2] * b[2]

    col = lambda j: [m[j], m[3 + j], m[6 + j]]
    c0, c1, c2 = col(0), col(1), col(2)
    S = {
        (0, 0): dot3(c0, c0), (1, 1): dot3(c1, c1), (2, 2): dot3(c2, c2),
        (0, 1): dot3(c0, c1), (0, 2): dot3(c0, c2), (1, 2): dot3(c1, c2),
    }
    one = jnp.ones_like(S[(0, 0)])
    zero = jnp.zeros_like(S[(0, 0)])
    V = [[one, zero, zero], [zero, one, zero], [zero, zero, one]]

    def getS(i, j):
        return S[(i, j)] if i <= j else S[(j, i)]

    for _ in range(3):
        for (p, q) in ((0, 1), (0, 2), (1, 2)):
            app, aqq, apq = getS(p, p), getS(q, q), getS(p, q)
            small = jnp.abs(apq) < 1e-30
            denom = jnp.where(small, one, 2.0 * apq)
            theta = (aqq - app) / denom
            t2 = theta * theta + 1.0
            rt = t2 * _rsqrt(t2)
            sgn = jnp.sign(theta)
            t = jnp.where(sgn == 0.0, one, sgn) / (jnp.abs(theta) + rt)
            c = _rsqrt(1.0 + t * t)
            s = t * c
            c = jnp.where(small, one, c)
            s = jnp.where(small, zero, s)
            r = 3 - p - q
            arp, arq = getS(r, p), getS(r, q)
            S[(p, p)] = c * c * app - 2.0 * s * c * apq + s * s * aqq
            S[(q, q)] = s * s * app + 2.0 * s * c * apq + c * c * aqq
            S[(p, q)] = zero
            S[(r, p) if r <= p else (p, r)] = c * arp - s * arq
            S[(r, q) if r <= q else (q, r)] = s * arp + c * arq
            for row in range(3):
                vp, vq = V[row][p], V[row][q]
                V[row][p] = c * vp - s * vq
                V[row][q] = s * vp + c * vq

    l0, l1, l2 = getS(0, 0), getS(1, 1), getS(2, 2)
    colV = lambda j: [V[0][j], V[1][j], V[2][j]]
    v0, v1, v2 = colV(0), colV(1), colV(2)
    is_min0 = (l0 <= l1) & (l0 <= l2)
    is_min2 = (~is_min0) & ((l2 <= l0) & (l2 <= l1))
    sel = lambda mask, a, b: [jnp.where(mask, x, y) for x, y in zip(a, b)]
    va = sel(is_min0, v1, v0)
    vb = sel(is_min2, v1, v2)

    def matvec(v):
        return [
            m[0] * v[0] + m[1] * v[1] + m[2] * v[2],
            m[3] * v[0] + m[4] * v[1] + m[5] * v[2],
            m[6] * v[0] + m[7] * v[1] + m[8] * v[2],
        ]

    ba, bb = matvec(va), matvec(vb)
    ua = [x * _rsqrt(dot3(ba, ba)) for x in ba]
    proj = dot3(ua, bb)
    ub = [x - proj * u for x, u in zip(bb, ua)]
    ub = [x * _rsqrt(dot3(ub, ub)) for x in ub]

    def cross(a, b):
        return [
            a[1] * b[2] - a[2] * b[1],
            a[2] * b[0] - a[0] * b[2],
            a[0] * b[1] - a[1] * b[0],
        ]

    uc = cross(ua, ub)
    wc = cross(va, vb)
    return [
        ua[r] * va[c] + ub[r] * vb[c] + uc[r] * wc[c]
        for r in range(3)
        for c in range(3)
    ]


@functools.partial(
    pl.kernel,
    mesh=plsc.VectorSubcoreMesh(core_axis_name="c", subcore_axis_name="s"),
    compiler_params=pltpu.CompilerParams(
        needs_layout_passes=False, use_tc_tiling_on_sc=True),
    out_type=(
        jax.ShapeDtypeStruct((9 * _NT, _B, 128), jnp.float32),
        jax.ShapeDtypeStruct((3 * _NT, _B, 128), jnp.float32),
    ),
    scratch_types=(
        [pltpu.VMEM((_N,), jnp.float32) for _ in range(9)]
        + [pltpu.VMEM((_N,), jnp.float32) for _ in range(3)]
        + [pltpu.VMEM((3, 3, _KH, _CH), jnp.float32),
           pltpu.VMEM((3, 3, _KH, _CH), jnp.float32),
           pltpu.VMEM((3, _KH, _CH), jnp.float32),
           pltpu.VMEM((3, _KH, _CH), jnp.float32),
           pltpu.VMEM((_KH, _CH), jnp.float32),
           pltpu.VMEM((_KH, _CH), jnp.float32),
           pltpu.VMEM((_KH, _CH), jnp.int32),
           pltpu.VMEM((_KH, _CH), jnp.int32),
           pltpu.VMEM((9, _NT // _PARTS, 128), jnp.float32),
           pltpu.VMEM((3, _NT // _PARTS, 128), jnp.float32),
           pltpu.VMEM((13, _CH), jnp.float32),
           pltpu.SemaphoreType.DMA,
           pltpu.SemaphoreType.DMA,
           ]
    ),
)
def _sc_solver(rot_hbm, trans_hbm, prot_hbm, ptrans_hbm, conf_hbm, topo_hbm,
               orot_hbm, otrans_hbm, *scratch):
    rot_v = scratch[0:9]
    trans_v = scratch[9:12]
    prot_ab = scratch[12:14]
    ptrans_ab = scratch[14:16]
    conf_ab = scratch[16:18]
    topo_ab = scratch[18:20]
    orot_v, otrans_v, acc_v = scratch[20:23]
    sems = scratch[23:25]

    wid = lax.axis_index("s") * _NC + lax.axis_index("c")
    b = wid // _PARTS
    part = wid % _PARTS
    nbase = part * _NPW

    for r in range(3):
        for c in range(3):
            pltpu.sync_copy(rot_hbm.at[r, c, b], rot_v[r * 3 + c])
    for c in range(3):
        pltpu.sync_copy(trans_hbm.at[c, b], trans_v[c])

    for j in range(9):
        def round_body(i, carry, _ref=rot_v[j]):
            s = pl.ds(i * _L, _L)
            _ref[s] = _bf16r(_ref[s])
            return carry

        lax.fori_loop(0, _N // _L, round_body, 0)

    def chunk_descs(ci, kh):
        nc = nbase + ci * _CH
        k0 = kh * _KH
        return (
            (prot_hbm.at[b, :, :, pl.ds(k0, _KH), pl.ds(nc, _CH)],
             prot_ab[kh]),
            (ptrans_hbm.at[b, :, pl.ds(k0, _KH), pl.ds(nc, _CH)],
             ptrans_ab[kh]),
            (conf_hbm.at[pl.ds(b * _K + k0, _KH), nc // 128], conf_ab[kh]),
            (topo_hbm.at[b, pl.ds(k0, _KH), pl.ds(nc, _CH)], topo_ab[kh]),
        )

    def start_half(ci, kh):
        for src, dst in chunk_descs(ci, kh):
            pltpu.make_async_copy(src, dst, sems[kh]).start()

    def wait_half(ci, kh):
        for src, dst in chunk_descs(ci, kh):
            pltpu.make_async_copy(src, dst, sems[kh]).wait()

    start_half(0, 0)
    start_half(0, 1)

    def accumulate(kh, g, init):
        prot_v, ptrans_v = prot_ab[kh], ptrans_ab[kh]
        conf_v, topo_v = conf_ab[kh], topo_ab[kh]
        nn = g * _L

        def k_body(k, acc):
            a9, a3, wsum = acc[:9], acc[9:12], acc[12]
            sl = pl.ds(nn, _L)
            t = topo_v[k, sl]
            w = conf_v[k, sl]
            Rj = [plsc.load_gather(rot_v[j], [t]) for j in range(9)]
            tj = [plsc.load_gather(trans_v[j], [t]) for j in range(3)]
            pr = [_bf16r(prot_v[j // 3, j % 3, k, sl]) for j in range(9)]
            pt = [_bf16r(ptrans_v[j, k, sl]) for j in range(3)]
            na9 = []
            for r in range(3):
                for c in range(3):
                    comp = (Rj[r * 3] * pr[c] + Rj[r * 3 + 1] * pr[3 + c]
                            + Rj[r * 3 + 2] * pr[6 + c])
                    na9.append(a9[r * 3 + c] + w * comp)
            na3 = []
            for r in range(3):
                ct = (Rj[r * 3] * pt[0] + Rj[r * 3 + 1] * pt[1]
                      + Rj[r * 3 + 2] * pt[2] + tj[r])
                na3.append(a3[r] + w * ct)
            return tuple(na9) + tuple(na3) + (wsum + w,)

        return lax.fori_loop(0, _KH, k_body, init, unroll=4)

    def chunk_loop(ci, carry):
        wait_half(ci, 0)

        def group_a(g, carry2):
            zero = jnp.zeros((_L,), jnp.float32)
            acc = accumulate(0, g, tuple(zero for _ in range(13)))
            for j in range(13):
                acc_v[j, pl.ds(g * _L, _L)] = acc[j]
            return carry2

        lax.fori_loop(0, _G, group_a, 0)

        @pl.when(ci < _NCHUNK - 1)
        def _():
            start_half(ci + 1, 0)

        wait_half(ci, 1)

        def group_b(g, carry2):
            init = tuple(acc_v[j, pl.ds(g * _L, _L)] for j in range(13))
            acc = accumulate(1, g, init)
            inv = 1.0 / acc[12]
            avg = [a * inv for a in acc[:9]]
            avgt = [a * inv for a in acc[9:12]]
            R = _proj_so3(avg)
            sl16 = pl.ds(g * _L, _L)
            for j in range(9):
                orot_v[j, ci, sl16] = R[j]
            for j in range(3):
                otrans_v[j, ci, sl16] = avgt[j]
            return carry2

        lax.fori_loop(0, _G, group_b, 0)

        @pl.when(ci < _NCHUNK - 1)
        def _():
            start_half(ci + 1, 1)

        return carry

    lax.fori_loop(0, _NCHUNK, chunk_loop, 0)

    row0 = part * (_NT // _PARTS)
    for j in range(9):
        pltpu.sync_copy(orot_v.at[j],
                        orot_hbm.at[pl.ds(j * _NT + row0, _NT // _PARTS), b])
    for j in range(3):
        pltpu.sync_copy(otrans_v.at[j],
                        otrans_hbm.at[pl.ds(j * _NT + row0, _NT // _PARTS), b])


def kernel(rot, trans, pair_rot, pair_trans, confidences, topology):
    rot_t = jnp.transpose(rot, (2, 3, 0, 1))
    trans_t = jnp.transpose(trans, (2, 0, 1))
    prot_t = jnp.transpose(pair_rot, (0, 3, 4, 2, 1))
    ptrans_t = jnp.transpose(pair_trans, (0, 3, 2, 1))
    conf_t = jnp.transpose(confidences, (0, 2, 3, 1)).reshape(
        _B * _K, _N // 128, 128)
    topo_t = jnp.transpose(topology.astype(jnp.int32), (0, 2, 1))
    orot, otrans = _sc_solver(rot_t, trans_t, prot_t, ptrans_t, conf_t, topo_t)
    out_rot = orot.reshape(3, 3, _NT, _B, 128).transpose(
        (3, 2, 4, 0, 1)).reshape(_B, _N, 3, 3)
    out_trans = otrans.reshape(3, _NT, _B, 128).transpose(
        (2, 1, 3, 0)).reshape(_B, _N, 3)
    return out_rot, out_trans


# speedup vs baseline: 41.0408x; 1.5680x over previous
import functools

import jax
import jax.numpy as jnp
from jax import lax
from jax.experimental import pallas as pl
from jax.experimental.pallas import tpu as pltpu
from jax.experimental.pallas import tpu_sc as plsc

_B, _N, _K = 8, 4096, 32
_NC, _NS, _L = 2, 16, 16
_NW = _NC * _NS
_NPW = _B * _N // _NW
_CH = 128
_NCHUNK = _NPW // _CH
_G = _CH // _L
_KH = _K // 2
_PARTS = _NW // _B
_NT = _N // 128


def _bf16r(x):
    i = lax.bitcast_convert_type(x, jnp.int32)
    i = jnp.bitwise_and(i + jnp.int32(0x8000), jnp.int32(-65536))
    return lax.bitcast_convert_type(i, jnp.float32)


def _rsqrt(x):
    i = lax.bitcast_convert_type(x, jnp.int32)
    i = jnp.int32(0x5F3759DF) - jnp.right_shift(i, jnp.ones_like(i))
    y = lax.bitcast_convert_type(i, jnp.float32)
    for _ in range(3):
        y = y * (1.5 - 0.5 * x * y * y)
    return y


def _proj_so3(m):
    def dot3(a, b):
        return a[0] * b[0] + a[1] * b[1] + a[2] * b[2]

    col = lambda j: [m[j], m[3 + j], m[6 + j]]
    c0, c1, c2 = col(0), col(1), col(2)
    S = {
        (0, 0): dot3(c0, c0), (1, 1): dot3(c1, c1), (2, 2): dot3(c2, c2),
        (0, 1): dot3(c0, c1), (0, 2): dot3(c0, c2), (1, 2): dot3(c1, c2),
    }
    one = jnp.ones_like(S[(0, 0)])
    zero = jnp.zeros_like(S[(0, 0)])
    V = [[one, zero, zero], [zero, one, zero], [zero, zero, one]]

    def getS(i, j):
        return S[(i, j)] if i <= j else S[(j, i)]

    for _ in range(3):
        for (p, q) in ((0, 1), (0, 2), (1, 2)):
            app, aqq, apq = getS(p, p), getS(q, q), getS(p, q)
            small = jnp.abs(apq) < 1e-30
            denom = jnp.where(small, one, 2.0 * apq)
            theta = (aqq - app) / denom
            t2 = theta * theta + 1.0
            rt = t2 * _rsqrt(t2)
            sgn = jnp.sign(theta)
            t = jnp.where(sgn == 0.0, one, sgn) / (jnp.abs(theta) + rt)
            c = _rsqrt(1.0 + t * t)
            s = t * c
            c = jnp.where(small, one, c)
            s = jnp.where(small, zero, s)
            r = 3 - p - q
            arp, arq = getS(r, p), getS(r, q)
            S[(p, p)] = c * c * app - 2.0 * s * c * apq + s * s * aqq
            S[(q, q)] = s * s * app + 2.0 * s * c * apq + c * c * aqq
            S[(p, q)] = zero
            S[(r, p) if r <= p else (p, r)] = c * arp - s * arq
            S[(r, q) if r <= q else (q, r)] = s * arp + c * arq
            for row in range(3):
                vp, vq = V[row][p], V[row][q]
                V[row][p] = c * vp - s * vq
                V[row][q] = s * vp + c * vq

    l0, l1, l2 = getS(0, 0), getS(1, 1), getS(2, 2)
    colV = lambda j: [V[0][j], V[1][j], V[2][j]]
    v0, v1, v2 = colV(0), colV(1), colV(2)
    is_min0 = (l0 <= l1) & (l0 <= l2)
    is_min2 = (~is_min0) & ((l2 <= l0) & (l2 <= l1))
    sel = lambda mask, a, b: [jnp.where(mask, x, y) for x, y in zip(a, b)]
    va = sel(is_min0, v1, v0)
    vb = sel(is_min2, v1, v2)

    def matvec(v):
        return [
            m[0] * v[0] + m[1] * v[1] + m[2] * v[2],
            m[3] * v[0] + m[4] * v[1] + m[5] * v[2],
            m[6] * v[0] + m[7] * v[1] + m[8] * v[2],
        ]

    ba, bb = matvec(va), matvec(vb)
    ua = [x * _rsqrt(dot3(ba, ba)) for x in ba]
    proj = dot3(ua, bb)
    ub = [x - proj * u for x, u in zip(bb, ua)]
    ub = [x * _rsqrt(dot3(ub, ub)) for x in ub]

    def cross(a, b):
        return [
            a[1] * b[2] - a[2] * b[1],
            a[2] * b[0] - a[0] * b[2],
            a[0] * b[1] - a[1] * b[0],
        ]

    uc = cross(ua, ub)
    wc = cross(va, vb)
    return [
        ua[r] * va[c] + ub[r] * vb[c] + uc[r] * wc[c]
        for r in range(3)
        for c in range(3)
    ]


@functools.partial(
    pl.kernel,
    mesh=plsc.VectorSubcoreMesh(core_axis_name="c", subcore_axis_name="s"),
    compiler_params=pltpu.CompilerParams(
        needs_layout_passes=False, use_tc_tiling_on_sc=True),
    out_type=(
        jax.ShapeDtypeStruct((9 * _NT, _B, 128), jnp.float32),
        jax.ShapeDtypeStruct((3 * _NT, _B, 128), jnp.float32),
    ),
    scratch_types=(
        [pltpu.VMEM((_N,), jnp.float32) for _ in range(9)]
        + [pltpu.VMEM((_N,), jnp.float32) for _ in range(3)]
        + [pltpu.VMEM((3, 3, _KH, _CH), jnp.float32),
           pltpu.VMEM((3, 3, _KH, _CH), jnp.float32),
           pltpu.VMEM((3, _KH, _CH), jnp.float32),
           pltpu.VMEM((3, _KH, _CH), jnp.float32),
           pltpu.VMEM((_KH, _CH), jnp.float32),
           pltpu.VMEM((_KH, _CH), jnp.float32),
           pltpu.VMEM((_KH, _CH), jnp.int32),
           pltpu.VMEM((_KH, _CH), jnp.int32),
           pltpu.VMEM((9, _NT // _PARTS, 128), jnp.float32),
           pltpu.VMEM((3, _NT // _PARTS, 128), jnp.float32),
           pltpu.VMEM((13, _CH), jnp.float32),
           pltpu.SemaphoreType.DMA,
           pltpu.SemaphoreType.DMA,
           ]
    ),
)
def _sc_solver(rot_hbm, trans_hbm, prot_hbm, ptrans_hbm, conf_hbm, topo_hbm,
               orot_hbm, otrans_hbm, *scratch):
    rot_v = scratch[0:9]
    trans_v = scratch[9:12]
    prot_ab = scratch[12:14]
    ptrans_ab = scratch[14:16]
    conf_ab = scratch[16:18]
    topo_ab = scratch[18:20]
    orot_v, otrans_v, acc_v = scratch[20:23]
    sems = scratch[23:25]

    wid = lax.axis_index("s") * _NC + lax.axis_index("c")
    b = wid // _PARTS
    part = wid % _PARTS
    nbase = part * _NPW

    for r in range(3):
        for c in range(3):
            pltpu.sync_copy(rot_hbm.at[r, c, b], rot_v[r * 3 + c])
    for c in range(3):
        pltpu.sync_copy(trans_hbm.at[c, b], trans_v[c])

    for j in range(9):
        def round_body(i, carry, _ref=rot_v[j]):
            s = pl.ds(i * _L, _L)
            _ref[s] = _bf16r(_ref[s])
            return carry

        lax.fori_loop(0, _N // _L, round_body, 0)

    def chunk_descs(ci, kh):
        nc = nbase + ci * _CH
        k0 = kh * _KH
        return (
            (prot_hbm.at[b, :, :, pl.ds(k0, _KH), pl.ds(nc, _CH)],
             prot_ab[kh]),
            (ptrans_hbm.at[b, :, pl.ds(k0, _KH), pl.ds(nc, _CH)],
             ptrans_ab[kh]),
            (conf_hbm.at[pl.ds(b * _K + k0, _KH), nc // 128], conf_ab[kh]),
            (topo_hbm.at[b, pl.ds(k0, _KH), pl.ds(nc, _CH)], topo_ab[kh]),
        )

    def start_half(ci, kh):
        for src, dst in chunk_descs(ci, kh):
            pltpu.make_async_copy(src, dst, sems[kh]).start()

    def wait_half(ci, kh):
        for src, dst in chunk_descs(ci, kh):
            pltpu.make_async_copy(src, dst, sems[kh]).wait()

    start_half(0, 0)
    start_half(0, 1)

    def accumulate(kh, g, init):
        prot_v, ptrans_v = prot_ab[kh], ptrans_ab[kh]
        conf_v, topo_v = conf_ab[kh], topo_ab[kh]
        nn = g * _L

        def k_body(k, acc):
            a9, a3, wsum = acc[:9], acc[9:12], acc[12]
            sl = pl.ds(nn, _L)
            t = topo_v[k, sl]
            w = conf_v[k, sl]
            Rj = [plsc.load_gather(rot_v[j], [t]) for j in range(9)]
            tj = [plsc.load_gather(trans_v[j], [t]) for j in range(3)]
            pr = [_bf16r(prot_v[j // 3, j % 3, k, sl]) for j in range(9)]
            pt = [_bf16r(ptrans_v[j, k, sl]) for j in range(3)]
            na9 = []
            for r in range(3):
                for c in range(3):
                    comp = (Rj[r * 3] * pr[c] + Rj[r * 3 + 1] * pr[3 + c]
                            + Rj[r * 3 + 2] * pr[6 + c])
                    na9.append(a9[r * 3 + c] + w * comp)
            na3 = []
            for r in range(3):
                ct = (Rj[r * 3] * pt[0] + Rj[r * 3 + 1] * pt[1]
                      + Rj[r * 3 + 2] * pt[2] + tj[r])
                na3.append(a3[r] + w * ct)
            return tuple(na9) + tuple(na3) + (wsum + w,)

        return lax.fori_loop(0, _KH, k_body, init, unroll=8)

    def chunk_loop(ci, carry):
        wait_half(ci, 0)

        def group_a(g, carry2):
            zero = jnp.zeros((_L,), jnp.float32)
            acc = accumulate(0, g, tuple(zero for _ in range(13)))
            for j in range(13):
                acc_v[j, pl.ds(g * _L, _L)] = acc[j]
            return carry2

        lax.fori_loop(0, _G, group_a, 0)

        @pl.when(ci < _NCHUNK - 1)
        def _():
            start_half(ci + 1, 0)

        wait_half(ci, 1)

        def group_b(g, carry2):
            init = tuple(acc_v[j, pl.ds(g * _L, _L)] for j in range(13))
            acc = accumulate(1, g, init)
            inv = 1.0 / acc[12]
            avg = [a * inv for a in acc[:9]]
            avgt = [a * inv for a in acc[9:12]]
            R = _proj_so3(avg)
            sl16 = pl.ds(g * _L, _L)
            for j in range(9):
                orot_v[j, ci, sl16] = R[j]
            for j in range(3):
                otrans_v[j, ci, sl16] = avgt[j]
            return carry2

        lax.fori_loop(0, _G, group_b, 0)

        @pl.when(ci < _NCHUNK - 1)
        def _():
            start_half(ci + 1, 1)

        return carry

    lax.fori_loop(0, _NCHUNK, chunk_loop, 0)

    row0 = part * (_NT // _PARTS)
    for j in range(9):
        pltpu.sync_copy(orot_v.at[j],
                        orot_hbm.at[pl.ds(j * _NT + row0, _NT // _PARTS), b])
    for j in range(3):
        pltpu.sync_copy(otrans_v.at[j],
                        otrans_hbm.at[pl.ds(j * _NT + row0, _NT // _PARTS), b])


def kernel(rot, trans, pair_rot, pair_trans, confidences, topology):
    rot_t = jnp.transpose(rot, (2, 3, 0, 1))
    trans_t = jnp.transpose(trans, (2, 0, 1))
    prot_t = jnp.transpose(pair_rot, (0, 3, 4, 2, 1))
    ptrans_t = jnp.transpose(pair_trans, (0, 3, 2, 1))
    conf_t = jnp.transpose(confidences, (0, 2, 3, 1)).reshape(
        _B * _K, _N // 128, 128)
    topo_t = jnp.transpose(topology.astype(jnp.int32), (0, 2, 1))
    orot, otrans = _sc_solver(rot_t, trans_t, prot_t, ptrans_t, conf_t, topo_t)
    out_rot = orot.reshape(3, 3, _NT, _B, 128).transpose(
        (3, 2, 4, 0, 1)).reshape(_B, _N, 3, 3)
    out_trans = otrans.reshape(3, _NT, _B, 128).transpose(
        (2, 1, 3, 0)).reshape(_B, _N, 3)
    return out_rot, out_trans
